# baseline (speedup 1.0000x reference)
import jax
import jax.numpy as jnp
from jax.experimental import pallas as pl
from jax.experimental.pallas import tpu as pltpu

N = 2048
NFEAT = 512
NHID = 64
DHID1 = 64
NSLAB = 8
SROWS = N // NSLAB
DEC_ROWS = 256
NSTEPS = 18


def _bf(v):
    return v.astype(jnp.bfloat16)


def _dot(a, b):
    return jax.lax.dot(a, b, preferred_element_type=jnp.float32)


def _bdot(a, b):
    return jax.lax.dot(_bf(a), _bf(b), preferred_element_type=jnp.float32)


_SCALE = float(N)


def _f8(v):
    return v.astype(jnp.float8_e4m3fn)


def _mega_body(ap_ref, aa_ref, an_ref, x_ref, w1_ref, w2_ref,
               attw_ref, dw1_ref, db1_ref, dw2_ref, db2_ref, dw3_ref, db3_ref,
               aw1_ref, ab1_ref, aw2_ref, ab2_ref, aw3_ref, ab3_ref, dec_ref,
               y_ref, buf_ref, sem_ref, xb_ref, p_ref, o_ref, emb_ref, z_ref):
    g = pl.program_id(0)

    def _start(step, slot):
        tt = step // 3
        vv = step % 3
        for k, ar in enumerate((ap_ref, aa_ref, an_ref)):
            @pl.when(tt == k)
            def _():
                for j in range(NSLAB):
                    pltpu.make_async_copy(
                        ar.at[vv, pl.ds(j * SROWS, SROWS), :],
                        buf_ref.at[slot, j],
                        sem_ref.at[slot, j]).start()

    def _p_store(s):
        i = (s % 3) * 3 + s // 3
        ps = _dot(xb_ref[...], _bf(w1_ref[i]))
        p_ref[s] = _f8(ps)

    @pl.when(g == 0)
    def _():
        _start(0, 0)
        xb_ref[...] = _bf(x_ref[...])
        _p_store(0)

    @pl.when(g < 8)
    def _():
        _start(g + 1, jax.lax.rem(g + 1, 2))
        _p_store(g + 1)

    @pl.when(g < 9)
    def _():
        slot = jax.lax.rem(g, 2)
        i = (g % 3) * 3 + g // 3
        p = p_ref[g]
        slabs = []
        hs = []
        for j in range(NSLAB):
            pltpu.make_async_copy(
                ap_ref.at[0, pl.ds(j * SROWS, SROWS), :],
                buf_ref.at[slot, j],
                sem_ref.at[slot, j]).wait()
            s8 = _f8(buf_ref[slot, j] * _SCALE)
            slabs.append(s8)
            hs.append(_dot(s8, p))
        h = jnp.concatenate(hs, axis=0)
        h = jnp.maximum(h, 0.0) * (1.0 / _SCALE)
        q = _f8(_dot(_bf(h), _bf(w2_ref[i])) * _SCALE)
        for j in range(NSLAB):
            o_ref[g, j * SROWS:(j + 1) * SROWS, :] = _bf(
                _dot(slabs[j], q) * (1.0 / (_SCALE * _SCALE)))

    @pl.when(g == 9)
    def _():
        embs = []
        for v in range(3):
            w1 = dw1_ref[v]
            hh = jnp.maximum(
                _bdot(o_ref[0 + v], w1[0 * NHID:1 * NHID])
                + _bdot(o_ref[3 + v], w1[1 * NHID:2 * NHID])
                + _bdot(o_ref[6 + v], w1[2 * NHID:3 * NHID])
                + db1_ref[v:v + 1, :], 0.0)
            hh = jnp.maximum(_bdot(hh, dw2_ref[v]) + db2_ref[v:v + 1, :], 0.0)
            embs.append(_bdot(hh, dw3_ref[v]) + db3_ref[v:v + 1, :])
        main, e1, e2 = embs
        aw = attw_ref[...]
        m = jnp.max(aw, axis=1, keepdims=True)
        ex = jnp.exp(aw - m)
        s = ex / jnp.sum(ex, axis=1, keepdims=True)
        s1 = e1 * s[:, 0:1]
        s2 = e2 * s[:, 1:2]
        gg = jnp.maximum(
            _bdot(s1, aw1_ref[0:DHID1]) + _bdot(s2, aw1_ref[DHID1:2 * DHID1])
            + ab1_ref[...], 0.0)
        gg = jnp.maximum(_bdot(gg, aw2_ref[...]) + ab2_ref[...], 0.0)
        sagg = _bdot(gg, aw3_ref[...]) + ab3_ref[...]
        emb_ref[:, 0:DHID1] = _bf(main)
        emb_ref[:, DHID1:2 * DHID1] = _bf(sagg)
        z_ref[...] = _bf(_bdot(main, dec_ref[0:DHID1])
                         + _bdot(sagg, dec_ref[DHID1:2 * DHID1]))

    @pl.when(g >= 10)
    def _():
        i = g - 10
        zz = z_ref[pl.ds(i * DEC_ROWS, DEC_ROWS), :]
        logits = jax.lax.dot_general(
            zz, emb_ref[...], dimension_numbers=(((1,), (1,)), ((), ())),
            preferred_element_type=jnp.float32)
        y_ref[...] = 0.5 * (1.0 + jnp.tanh(0.5 * logits))


def kernel(x, adjs_pos, adjs_add, adjs_neg, attW, enc_W1, enc_W2,
           dsn_W1, dsn_b1, dsn_W2, dsn_b2, dsn_W3, dsn_b3,
           agg_W1, agg_b1, agg_W2, agg_b2, agg_W3, agg_b3, dec_W):
    w1_all = enc_W1.reshape(9, NFEAT, NHID)
    w2_all = enc_W2.reshape(9, NHID, NHID)

    def _c(spec_shape):
        return pl.BlockSpec(spec_shape, lambda g: tuple(0 for _ in spec_shape))

    y = pl.pallas_call(
        _mega_body,
        grid=(NSTEPS,),
        in_specs=[
            pl.BlockSpec(memory_space=pl.ANY),
            pl.BlockSpec(memory_space=pl.ANY),
            pl.BlockSpec(memory_space=pl.ANY),
            _c((N, NFEAT)),
            _c((9, NFEAT, NHID)),
            _c((9, NHID, NHID)),
            _c((1, 2)),
            _c((3, 3 * NHID, DHID1)),
            _c((3, DHID1)),
            _c((3, DHID1, 2 * DHID1)),
            _c((3, 2 * DHID1)),
            _c((3, 2 * DHID1, DHID1)),
            _c((3, DHID1)),
            _c((2 * DHID1, 2 * DHID1)),
            _c((1, 2 * DHID1)),
            _c((2 * DHID1, 4 * DHID1)),
            _c((1, 4 * DHID1)),
            _c((4 * DHID1, DHID1)),
            _c((1, DHID1)),
            _c((2 * DHID1, 2 * DHID1)),
        ],
        out_specs=pl.BlockSpec(
            (DEC_ROWS, N), lambda g: (jnp.maximum(g - 10, 0), 0)),
        out_shape=jax.ShapeDtypeStruct((N, N), jnp.float32),
        scratch_shapes=[
            pltpu.VMEM((2, NSLAB, SROWS, N), jnp.float32),
            pltpu.SemaphoreType.DMA((2, NSLAB)),
            pltpu.VMEM((N, NFEAT), jnp.bfloat16),
            pltpu.VMEM((9, N, NHID), jnp.float8_e4m3fn),
            pltpu.VMEM((9, N, NHID), jnp.bfloat16),
            pltpu.VMEM((N, 2 * DHID1), jnp.bfloat16),
            pltpu.VMEM((N, 2 * DHID1), jnp.bfloat16),
        ],
        compiler_params=pltpu.CompilerParams(
            vmem_limit_bytes=100 * 1024 * 1024,
        ),
    )(adjs_pos, adjs_add, adjs_neg, x, w1_all, w2_all,
      attW.reshape(1, 2), dsn_W1, dsn_b1, dsn_W2, dsn_b2, dsn_W3, dsn_b3,
      agg_W1, agg_b1.reshape(1, -1), agg_W2, agg_b2.reshape(1, -1),
      agg_W3, agg_b3.reshape(1, -1), dec_W)
    return y


# speedup vs baseline: 1.0209x; 1.0209x over previous
import jax
import jax.numpy as jnp
from jax.experimental import pallas as pl
from jax.experimental.pallas import tpu as pltpu

N = 2048
NFEAT = 512
NHID = 64
DHID1 = 64
NSLAB = 8
SROWS = N // NSLAB
DEC_ROWS = 256
NSTEPS = 18


def _bf(v):
    return v.astype(jnp.bfloat16)


def _dot(a, b):
    return jax.lax.dot(a, b, preferred_element_type=jnp.float32)


def _bdot(a, b):
    return jax.lax.dot(_bf(a), _bf(b), preferred_element_type=jnp.float32)


_SCALE = float(N)


def _f8(v):
    return v.astype(jnp.float8_e4m3fn)


def _mega_body(ap_ref, aa_ref, an_ref, x_ref, w1_ref, w2_ref,
               attw_ref, dw1_ref, db1_ref, dw2_ref, db2_ref, dw3_ref, db3_ref,
               aw1_ref, ab1_ref, aw2_ref, ab2_ref, aw3_ref, ab3_ref, dec_ref,
               y_ref, buf_ref, sem_ref, xb_ref, p_ref, o_ref, emb_ref, z_ref):
    g = pl.program_id(0)

    def _start(step, slot):
        tt = step // 3
        vv = step % 3
        for k, ar in enumerate((ap_ref, aa_ref, an_ref)):
            @pl.when(tt == k)
            def _():
                for j in range(NSLAB):
                    pltpu.make_async_copy(
                        ar.at[vv, pl.ds(j * SROWS, SROWS), :],
                        buf_ref.at[slot, j],
                        sem_ref.at[slot, j]).start()

    def _p_store(s):
        i = (s % 3) * 3 + s // 3
        ps = _dot(xb_ref[...], _bf(w1_ref[i]))
        p_ref[s] = _f8(ps)

    @pl.when(g == 0)
    def _():
        _start(0, 0)
        xb_ref[...] = _bf(x_ref[...])
        _p_store(0)

    @pl.when(g < 8)
    def _():
        _start(g + 1, jax.lax.rem(g + 1, 2))
        _p_store(g + 1)

    @pl.when(g < 9)
    def _():
        slot = jax.lax.rem(g, 2)
        for j in range(NSLAB):
            pltpu.make_async_copy(
                ap_ref.at[0, pl.ds(j * SROWS, SROWS), :],
                buf_ref.at[slot, j],
                sem_ref.at[slot, j]).wait()
        i = (g % 3) * 3 + g // 3
        p = p_ref[g]
        slabs = [_f8(buf_ref[slot, j] * _SCALE) for j in range(NSLAB)]
        h = jnp.concatenate(
            [_dot(s, p) for s in slabs], axis=0)
        h = jnp.maximum(h, 0.0) * (1.0 / _SCALE)
        q = _f8(_dot(_bf(h), _bf(w2_ref[i])) * _SCALE)
        for j in range(NSLAB):
            o_ref[g, j * SROWS:(j + 1) * SROWS, :] = _bf(
                _dot(slabs[j], q) * (1.0 / (_SCALE * _SCALE)))

    @pl.when(g == 9)
    def _():
        embs = []
        for v in range(3):
            w1 = dw1_ref[v]
            hh = jnp.maximum(
                _bdot(o_ref[0 + v], w1[0 * NHID:1 * NHID])
                + _bdot(o_ref[3 + v], w1[1 * NHID:2 * NHID])
                + _bdot(o_ref[6 + v], w1[2 * NHID:3 * NHID])
                + db1_ref[v:v + 1, :], 0.0)
            hh = jnp.maximum(_bdot(hh, dw2_ref[v]) + db2_ref[v:v + 1, :], 0.0)
            embs.append(_bdot(hh, dw3_ref[v]) + db3_ref[v:v + 1, :])
        main, e1, e2 = embs
        aw = attw_ref[...]
        m = jnp.max(aw, axis=1, keepdims=True)
        ex = jnp.exp(aw - m)
        s = ex / jnp.sum(ex, axis=1, keepdims=True)
        s1 = e1 * s[:, 0:1]
        s2 = e2 * s[:, 1:2]
        gg = jnp.maximum(
            _bdot(s1, aw1_ref[0:DHID1]) + _bdot(s2, aw1_ref[DHID1:2 * DHID1])
            + ab1_ref[...], 0.0)
        gg = jnp.maximum(_bdot(gg, aw2_ref[...]) + ab2_ref[...], 0.0)
        sagg = _bdot(gg, aw3_ref[...]) + ab3_ref[...]
        emb_ref[:, 0:DHID1] = _bf(main)
        emb_ref[:, DHID1:2 * DHID1] = _bf(sagg)
        z_ref[...] = _bf(_bdot(main, dec_ref[0:DHID1])
                         + _bdot(sagg, dec_ref[DHID1:2 * DHID1]))

    @pl.when(g >= 10)
    def _():
        i = g - 10
        zz = z_ref[pl.ds(i * DEC_ROWS, DEC_ROWS), :]
        logits = jax.lax.dot_general(
            zz, emb_ref[...], dimension_numbers=(((1,), (1,)), ((), ())),
            preferred_element_type=jnp.float32)
        y_ref[...] = 0.5 * (1.0 + jnp.tanh(0.5 * logits))


def kernel(x, adjs_pos, adjs_add, adjs_neg, attW, enc_W1, enc_W2,
           dsn_W1, dsn_b1, dsn_W2, dsn_b2, dsn_W3, dsn_b3,
           agg_W1, agg_b1, agg_W2, agg_b2, agg_W3, agg_b3, dec_W):
    w1_all = enc_W1.reshape(9, NFEAT, NHID)
    w2_all = enc_W2.reshape(9, NHID, NHID)

    def _c(spec_shape):
        return pl.BlockSpec(spec_shape, lambda g: tuple(0 for _ in spec_shape))

    y = pl.pallas_call(
        _mega_body,
        grid=(NSTEPS,),
        in_specs=[
            pl.BlockSpec(memory_space=pl.ANY),
            pl.BlockSpec(memory_space=pl.ANY),
            pl.BlockSpec(memory_space=pl.ANY),
            _c((N, NFEAT)),
            _c((9, NFEAT, NHID)),
            _c((9, NHID, NHID)),
            _c((1, 2)),
            _c((3, 3 * NHID, DHID1)),
            _c((3, DHID1)),
            _c((3, DHID1, 2 * DHID1)),
            _c((3, 2 * DHID1)),
            _c((3, 2 * DHID1, DHID1)),
            _c((3, DHID1)),
            _c((2 * DHID1, 2 * DHID1)),
            _c((1, 2 * DHID1)),
            _c((2 * DHID1, 4 * DHID1)),
            _c((1, 4 * DHID1)),
            _c((4 * DHID1, DHID1)),
            _c((1, DHID1)),
            _c((2 * DHID1, 2 * DHID1)),
        ],
        out_specs=pl.BlockSpec(
            (DEC_ROWS, N), lambda g: (jnp.maximum(g - 10, 0), 0)),
        out_shape=jax.ShapeDtypeStruct((N, N), jnp.float32),
        scratch_shapes=[
            pltpu.VMEM((2, NSLAB, SROWS, N), jnp.float32),
            pltpu.SemaphoreType.DMA((2, NSLAB)),
            pltpu.VMEM((N, NFEAT), jnp.bfloat16),
            pltpu.VMEM((9, N, NHID), jnp.float8_e4m3fn),
            pltpu.VMEM((9, N, NHID), jnp.bfloat16),
            pltpu.VMEM((N, 2 * DHID1), jnp.bfloat16),
            pltpu.VMEM((N, 2 * DHID1), jnp.bfloat16),
        ],
        compiler_params=pltpu.CompilerParams(
            vmem_limit_bytes=100 * 1024 * 1024,
        ),
    )(adjs_pos, adjs_add, adjs_neg, x, w1_all, w2_all,
      attW.reshape(1, 2), dsn_W1, dsn_b1, dsn_W2, dsn_b2, dsn_W3, dsn_b3,
      agg_W1, agg_b1.reshape(1, -1), agg_W2, agg_b2.reshape(1, -1),
      agg_W3, agg_b3.reshape(1, -1), dec_W)
    return y


# speedup vs baseline: 1.0393x; 1.0181x over previous
import jax
import jax.numpy as jnp
from jax.experimental import pallas as pl
from jax.experimental.pallas import tpu as pltpu

N = 2048
NFEAT = 512
NHID = 64
DHID1 = 64
NSLAB = 8
SROWS = N // NSLAB
DEC_ROWS = 512
NSTEPS = 14


def _bf(v):
    return v.astype(jnp.bfloat16)


def _dot(a, b):
    return jax.lax.dot(a, b, preferred_element_type=jnp.float32)


def _bdot(a, b):
    return jax.lax.dot(_bf(a), _bf(b), preferred_element_type=jnp.float32)


_SCALE = float(N)


def _f8(v):
    return v.astype(jnp.float8_e4m3fn)


def _mega_body(ap_ref, aa_ref, an_ref, x_ref, w1_ref, w2_ref,
               attw_ref, dw1_ref, db1_ref, dw2_ref, db2_ref, dw3_ref, db3_ref,
               aw1_ref, ab1_ref, aw2_ref, ab2_ref, aw3_ref, ab3_ref, dec_ref,
               y_ref, buf_ref, sem_ref, xb_ref, p_ref, o_ref, emb_ref, z_ref):
    g = pl.program_id(0)

    def _start(step, slot):
        tt = step // 3
        vv = step % 3
        for k, ar in enumerate((ap_ref, aa_ref, an_ref)):
            @pl.when(tt == k)
            def _():
                for j in range(NSLAB):
                    pltpu.make_async_copy(
                        ar.at[vv, pl.ds(j * SROWS, SROWS), :],
                        buf_ref.at[slot, j],
                        sem_ref.at[slot, j]).start()

    def _p_store(s):
        i = (s % 3) * 3 + s // 3
        ps = _dot(xb_ref[...], _bf(w1_ref[i]))
        p_ref[s] = _f8(ps)

    @pl.when(g == 0)
    def _():
        _start(0, 0)
        xb_ref[...] = _bf(x_ref[...])
        _p_store(0)

    @pl.when(g < 8)
    def _():
        _start(g + 1, jax.lax.rem(g + 1, 2))
        _p_store(g + 1)

    @pl.when(g < 9)
    def _():
        slot = jax.lax.rem(g, 2)
        for j in range(NSLAB):
            pltpu.make_async_copy(
                ap_ref.at[0, pl.ds(j * SROWS, SROWS), :],
                buf_ref.at[slot, j],
                sem_ref.at[slot, j]).wait()
        i = (g % 3) * 3 + g // 3
        p = p_ref[g]
        slabs = [_f8(buf_ref[slot, j] * _SCALE) for j in range(NSLAB)]
        h = jnp.concatenate(
            [_dot(s, p) for s in slabs], axis=0)
        h = jnp.maximum(h, 0.0) * (1.0 / _SCALE)
        q = _f8(_dot(_bf(h), _bf(w2_ref[i])) * _SCALE)
        for j in range(NSLAB):
            o_ref[g, j * SROWS:(j + 1) * SROWS, :] = _bf(
                _dot(slabs[j], q) * (1.0 / (_SCALE * _SCALE)))

    @pl.when(g == 9)
    def _():
        embs = []
        for v in range(3):
            w1 = dw1_ref[v]
            hh = jnp.maximum(
                _bdot(o_ref[0 + v], w1[0 * NHID:1 * NHID])
                + _bdot(o_ref[3 + v], w1[1 * NHID:2 * NHID])
                + _bdot(o_ref[6 + v], w1[2 * NHID:3 * NHID])
                + db1_ref[v:v + 1, :], 0.0)
            hh = jnp.maximum(_bdot(hh, dw2_ref[v]) + db2_ref[v:v + 1, :], 0.0)
            embs.append(_bdot(hh, dw3_ref[v]) + db3_ref[v:v + 1, :])
        main, e1, e2 = embs
        aw = attw_ref[...]
        m = jnp.max(aw, axis=1, keepdims=True)
        ex = jnp.exp(aw - m)
        s = ex / jnp.sum(ex, axis=1, keepdims=True)
        s1 = e1 * s[:, 0:1]
        s2 = e2 * s[:, 1:2]
        gg = jnp.maximum(
            _bdot(s1, aw1_ref[0:DHID1]) + _bdot(s2, aw1_ref[DHID1:2 * DHID1])
            + ab1_ref[...], 0.0)
        gg = jnp.maximum(_bdot(gg, aw2_ref[...]) + ab2_ref[...], 0.0)
        sagg = _bdot(gg, aw3_ref[...]) + ab3_ref[...]
        emb_ref[:, 0:DHID1] = _bf(main)
        emb_ref[:, DHID1:2 * DHID1] = _bf(sagg)
        z_ref[...] = _bf(_bdot(main, dec_ref[0:DHID1])
                         + _bdot(sagg, dec_ref[DHID1:2 * DHID1]))

    @pl.when(g >= 10)
    def _():
        i = g - 10
        zz = z_ref[pl.ds(i * DEC_ROWS, DEC_ROWS), :]
        logits = jax.lax.dot_general(
            zz, emb_ref[...], dimension_numbers=(((1,), (1,)), ((), ())),
            preferred_element_type=jnp.float32)
        y_ref[...] = 0.5 * (1.0 + jnp.tanh(0.5 * logits))


def kernel(x, adjs_pos, adjs_add, adjs_neg, attW, enc_W1, enc_W2,
           dsn_W1, dsn_b1, dsn_W2, dsn_b2, dsn_W3, dsn_b3,
           agg_W1, agg_b1, agg_W2, agg_b2, agg_W3, agg_b3, dec_W):
    w1_all = enc_W1.reshape(9, NFEAT, NHID)
    w2_all = enc_W2.reshape(9, NHID, NHID)

    def _c(spec_shape):
        return pl.BlockSpec(spec_shape, lambda g: tuple(0 for _ in spec_shape))

    y = pl.pallas_call(
        _mega_body,
        grid=(NSTEPS,),
        in_specs=[
            pl.BlockSpec(memory_space=pl.ANY),
            pl.BlockSpec(memory_space=pl.ANY),
            pl.BlockSpec(memory_space=pl.ANY),
            _c((N, NFEAT)),
            _c((9, NFEAT, NHID)),
            _c((9, NHID, NHID)),
            _c((1, 2)),
            _c((3, 3 * NHID, DHID1)),
            _c((3, DHID1)),
            _c((3, DHID1, 2 * DHID1)),
            _c((3, 2 * DHID1)),
            _c((3, 2 * DHID1, DHID1)),
            _c((3, DHID1)),
            _c((2 * DHID1, 2 * DHID1)),
            _c((1, 2 * DHID1)),
            _c((2 * DHID1, 4 * DHID1)),
            _c((1, 4 * DHID1)),
            _c((4 * DHID1, DHID1)),
            _c((1, DHID1)),
            _c((2 * DHID1, 2 * DHID1)),
        ],
        out_specs=pl.BlockSpec(
            (DEC_ROWS, N), lambda g: (jnp.maximum(g - 10, 0), 0)),
        out_shape=jax.ShapeDtypeStruct((N, N), jnp.float32),
        scratch_shapes=[
            pltpu.VMEM((2, NSLAB, SROWS, N), jnp.float32),
            pltpu.SemaphoreType.DMA((2, NSLAB)),
            pltpu.VMEM((N, NFEAT), jnp.bfloat16),
            pltpu.VMEM((9, N, NHID), jnp.float8_e4m3fn),
            pltpu.VMEM((9, N, NHID), jnp.bfloat16),
            pltpu.VMEM((N, 2 * DHID1), jnp.bfloat16),
            pltpu.VMEM((N, 2 * DHID1), jnp.bfloat16),
        ],
        compiler_params=pltpu.CompilerParams(
            vmem_limit_bytes=100 * 1024 * 1024,
        ),
    )(adjs_pos, adjs_add, adjs_neg, x, w1_all, w2_all,
      attW.reshape(1, 2), dsn_W1, dsn_b1, dsn_W2, dsn_b2, dsn_W3, dsn_b3,
      agg_W1, agg_b1.reshape(1, -1), agg_W2, agg_b2.reshape(1, -1),
      agg_W3, agg_b3.reshape(1, -1), dec_W)
    return y
